# baseline (speedup 1.0000x reference)
import functools

import jax
import jax.numpy as jnp
from jax import lax
from jax.experimental import pallas as pl
from jax.experimental.pallas import tpu as pltpu
from jax.experimental.pallas import tpu_sc as plsc

N_ROWS = 16384
N_COLS = 832
N_FIELDS_K = 26
FIELD = 32
LANES = 16

NW = 32
ROWS_PER_W = N_ROWS // NW
CHUNK = 64
N_CHUNKS = ROWS_PER_W // CHUNK
BLOCKS = CHUNK // LANES
CHUNK_ELEMS = CHUNK * N_COLS


def _sc_body(x_hbm, out_hbm, ibuf, obuf, bidx_store):
    wid = lax.axis_index("s") * 2 + lax.axis_index("c")
    ji = lax.iota(jnp.int32, LANES)
    ones = jnp.full((LANES,), 1.0, jnp.float32)
    zeros = jnp.zeros((LANES,), jnp.float32)

    def zero_body(i, _):
        obuf[pl.ds(i * LANES, LANES)] = zeros
        return 0

    lax.fori_loop(0, CHUNK_ELEMS // LANES, zero_body, 0)

    def chunk_body(k, _):
        base = (wid * ROWS_PER_W + k * CHUNK) * N_COLS
        pltpu.sync_copy(x_hbm.at[pl.ds(base, CHUNK_ELEMS)], ibuf)

        def block_body(b, _):
            rbase = (b * LANES + ji) * N_COLS

            def seg_body(s, _):
                c0 = s * FIELD
                best = plsc.load_gather(ibuf, [rbase + c0])
                bcol = jnp.full((LANES,), 0, jnp.int32) + c0
                col = bcol
                for _step in range(1, FIELD):
                    col = col + 1
                    v = plsc.load_gather(ibuf, [rbase + col])
                    take = v > best
                    best = jnp.where(take, v, best)
                    bcol = jnp.where(take, col, bcol)
                plsc.store_scatter(obuf, [rbase + bcol], ones)
                bidx_store[pl.ds((b * N_FIELDS_K + s) * LANES, LANES)] = (
                    rbase + bcol)
                return 0

            lax.fori_loop(0, N_FIELDS_K, seg_body, 0)
            return 0

        lax.fori_loop(0, BLOCKS, block_body, 0)
        pltpu.sync_copy(obuf, out_hbm.at[pl.ds(base, CHUNK_ELEMS)])

        def clear_body(j, _):
            idx = bidx_store[pl.ds(j * LANES, LANES)]
            plsc.store_scatter(obuf, [idx], zeros)
            return 0

        lax.fori_loop(0, BLOCKS * N_FIELDS_K, clear_body, 0)
        return 0

    lax.fori_loop(0, N_CHUNKS, chunk_body, 0)


@jax.jit
def kernel(x):
    mesh = plsc.VectorSubcoreMesh(core_axis_name="c", subcore_axis_name="s")
    f = functools.partial(
        pl.kernel,
        mesh=mesh,
        out_type=jax.ShapeDtypeStruct((N_ROWS * N_COLS,), jnp.float32),
        scratch_types=[
            pltpu.VMEM((CHUNK_ELEMS,), jnp.float32),
            pltpu.VMEM((CHUNK_ELEMS,), jnp.float32),
            pltpu.VMEM((BLOCKS * N_FIELDS_K * LANES,), jnp.int32),
        ],
        compiler_params=pltpu.CompilerParams(needs_layout_passes=False),
    )(_sc_body)
    return f(x.reshape(-1)).reshape(N_ROWS, N_COLS)


# speedup vs baseline: 1.7094x; 1.7094x over previous
import functools

import jax
import jax.numpy as jnp
from jax import lax
from jax.experimental import pallas as pl
from jax.experimental.pallas import tpu as pltpu
from jax.experimental.pallas import tpu_sc as plsc

N_ROWS = 16384
N_COLS = 832
N_FIELDS_K = 26
FIELD = 32
LANES = 16

NW = 32
N_PARTS = 4
PART_ROWS = N_ROWS // N_PARTS
ROWS_PER_W = PART_ROWS // NW
CHUNK = 64
N_CHUNKS = ROWS_PER_W // CHUNK


def _sc_body(x_hbm, out_hbm, buf):
    wid = lax.axis_index("s") * 2 + lax.axis_index("c")
    ji = lax.iota(jnp.int32, LANES)

    def chunk_body(k, _):
        base = wid * ROWS_PER_W + k * CHUNK
        pltpu.sync_copy(x_hbm.at[pl.ds(base, CHUNK)], buf)

        def row_body(r, _):
            for f in range(N_FIELDS_K):
                c = f * FIELD
                v0 = buf[r, pl.ds(c, LANES)]
                v1 = buf[r, pl.ds(c + LANES, LANES)]
                m = jnp.max(jnp.maximum(v0, v1))
                eq0 = v0 == m
                eq1 = v1 == m
                n0 = plsc.all_reduce_population_count(eq0)
                f0 = plsc.all_reduce_ffs(eq0)
                f1 = plsc.all_reduce_ffs(eq1)
                first = jnp.where(n0 > 0, f0, f1 + LANES)
                buf[r, pl.ds(c, LANES)] = jnp.where(
                    ji == first, 1.0, 0.0).astype(jnp.float32)
                buf[r, pl.ds(c + LANES, LANES)] = jnp.where(
                    ji == first - LANES, 1.0, 0.0).astype(jnp.float32)
            return 0

        lax.fori_loop(0, CHUNK, row_body, 0)
        pltpu.sync_copy(buf, out_hbm.at[pl.ds(base, CHUNK)])
        return 0

    lax.fori_loop(0, N_CHUNKS, chunk_body, 0)


@jax.jit
def kernel(x):
    mesh = plsc.VectorSubcoreMesh(core_axis_name="c", subcore_axis_name="s")
    f = functools.partial(
        pl.kernel,
        mesh=mesh,
        out_type=jax.ShapeDtypeStruct((PART_ROWS, N_COLS), jnp.float32),
        scratch_types=[pltpu.VMEM((CHUNK, N_COLS), jnp.float32)],
        compiler_params=pltpu.CompilerParams(needs_layout_passes=False),
    )(_sc_body)
    parts = [
        f(lax.slice_in_dim(x, p * PART_ROWS, (p + 1) * PART_ROWS, axis=0))
        for p in range(N_PARTS)
    ]
    return jnp.concatenate(parts, axis=0)


# speedup vs baseline: 1.8631x; 1.0899x over previous
import functools

import jax
import jax.numpy as jnp
from jax import lax
from jax.experimental import pallas as pl
from jax.experimental.pallas import tpu as pltpu
from jax.experimental.pallas import tpu_sc as plsc

N_ROWS = 16384
N_COLS = 832
N_FIELDS_K = 26
FIELD = 32
LANES = 16

NW = 32
N_PARTS = 4
PART_ROWS = N_ROWS // N_PARTS
ROWS_PER_W = PART_ROWS // NW
CHUNK = 64
N_CHUNKS = ROWS_PER_W // CHUNK


def _sc_body(x_hbm, out_hbm, buf):
    wid = lax.axis_index("s") * 2 + lax.axis_index("c")
    ji = lax.iota(jnp.int32, LANES)

    def chunk_body(k, _):
        base = wid * ROWS_PER_W + k * CHUNK
        pltpu.sync_copy(x_hbm.at[pl.ds(base, CHUNK)], buf)

        def row_body(r, _):
            for f in range(N_FIELDS_K):
                c = f * FIELD
                v0 = buf[r, pl.ds(c, LANES)]
                v1 = buf[r, pl.ds(c + LANES, LANES)]
                m = jnp.max(jnp.maximum(v0, v1))
                eq0 = v0 == m
                eq1 = v1 == m
                n0 = plsc.all_reduce_population_count(eq0)
                f0 = plsc.all_reduce_ffs(eq0)
                f1 = plsc.all_reduce_ffs(eq1)
                first = jnp.where(n0 > 0, f0, f1 + LANES)
                buf[r, pl.ds(c, LANES)] = jnp.where(
                    ji == first, 1.0, 0.0).astype(jnp.float32)
                buf[r, pl.ds(c + LANES, LANES)] = jnp.where(
                    ji == first - LANES, 1.0, 0.0).astype(jnp.float32)
            return 0

        lax.fori_loop(0, CHUNK, row_body, 0)
        pltpu.sync_copy(buf, out_hbm.at[pl.ds(base, CHUNK)])
        return 0

    lax.fori_loop(0, N_CHUNKS, chunk_body, 0)


@jax.jit
def kernel(x):
    mesh = plsc.VectorSubcoreMesh(core_axis_name="c", subcore_axis_name="s")
    f = functools.partial(
        pl.kernel,
        mesh=mesh,
        out_type=jax.ShapeDtypeStruct((PART_ROWS, N_COLS), jnp.float32),
        scratch_types=[pltpu.VMEM((CHUNK, N_COLS), jnp.float32)],
        compiler_params=pltpu.CompilerParams(needs_layout_passes=False),
    )(_sc_body)
    parts = [
        f(lax.slice_in_dim(x, p * PART_ROWS, (p + 1) * PART_ROWS, axis=0))
        for p in range(N_PARTS)
    ]
    y = jnp.zeros((N_ROWS, N_COLS), jnp.float32)
    for p in range(N_PARTS):
        y = lax.dynamic_update_slice(y, parts[p], (p * PART_ROWS, 0))
    return y


# speedup vs baseline: 2.1912x; 1.1761x over previous
import functools

import jax
import jax.numpy as jnp
from jax import lax
from jax.experimental import pallas as pl
from jax.experimental.pallas import tpu as pltpu
from jax.experimental.pallas import tpu_sc as plsc

N_ROWS = 16384
N_COLS = 832
N_FIELDS_K = 26
FIELD = 32
LANES = 16

NW = 32
ROWS_PER_W = N_ROWS // NW
CHUNK = 32
N_CHUNKS = ROWS_PER_W // CHUNK


def _sc_body(x_hbm, out_hbm, in_a, in_b, out_a, out_b,
             s_ia, s_ib, s_oa, s_ob):
    wid = lax.axis_index("s") * 2 + lax.axis_index("c")
    ji = lax.iota(jnp.int32, LANES)
    row0 = wid * ROWS_PER_W

    def in_copy(k, buf, sem):
        return pltpu.make_async_copy(
            x_hbm.at[pl.ds(row0 + k * CHUNK, CHUNK)], buf, sem)

    def out_copy(k, buf, sem):
        return pltpu.make_async_copy(
            buf, out_hbm.at[pl.ds(row0 + k * CHUNK, CHUNK)], sem)

    def compute(ibuf, obuf):
        def row_body(r, _):
            for f in range(N_FIELDS_K):
                c = f * FIELD
                v0 = ibuf[r, pl.ds(c, LANES)]
                v1 = ibuf[r, pl.ds(c + LANES, LANES)]
                m = jnp.max(jnp.maximum(v0, v1))
                eq0 = v0 == m
                eq1 = v1 == m
                n0 = plsc.all_reduce_population_count(eq0)
                f0 = plsc.all_reduce_ffs(eq0)
                f1 = plsc.all_reduce_ffs(eq1)
                first = jnp.where(n0 > 0, f0, f1 + LANES)
                obuf[r, pl.ds(c, LANES)] = jnp.where(
                    ji == first, 1.0, 0.0).astype(jnp.float32)
                obuf[r, pl.ds(c + LANES, LANES)] = jnp.where(
                    ji == first - LANES, 1.0, 0.0).astype(jnp.float32)
            return 0

        lax.fori_loop(0, CHUNK, row_body, 0)

    in_copy(0, in_a, s_ia).start()
    in_copy(1, in_b, s_ib).start()

    in_copy(0, in_a, s_ia).wait()
    compute(in_a, out_a)
    out_copy(0, out_a, s_oa).start()
    in_copy(2, in_a, s_ia).start()

    in_copy(1, in_b, s_ib).wait()
    compute(in_b, out_b)
    out_copy(1, out_b, s_ob).start()
    in_copy(3, in_b, s_ib).start()

    def pair_body(jj, _):
        k0 = 2 * jj
        k1 = k0 + 1
        in_copy(k0, in_a, s_ia).wait()
        out_copy(k0 - 2, out_a, s_oa).wait()
        compute(in_a, out_a)
        out_copy(k0, out_a, s_oa).start()
        in_copy(k0 + 2, in_a, s_ia).start()

        in_copy(k1, in_b, s_ib).wait()
        out_copy(k1 - 2, out_b, s_ob).wait()
        compute(in_b, out_b)
        out_copy(k1, out_b, s_ob).start()
        in_copy(k1 + 2, in_b, s_ib).start()
        return 0

    lax.fori_loop(1, N_CHUNKS // 2 - 1, pair_body, 0)

    kl = N_CHUNKS - 2
    in_copy(kl, in_a, s_ia).wait()
    out_copy(kl - 2, out_a, s_oa).wait()
    compute(in_a, out_a)
    out_copy(kl, out_a, s_oa).start()

    in_copy(kl + 1, in_b, s_ib).wait()
    out_copy(kl - 1, out_b, s_ob).wait()
    compute(in_b, out_b)
    out_copy(kl + 1, out_b, s_ob).start()

    out_copy(kl, out_a, s_oa).wait()
    out_copy(kl + 1, out_b, s_ob).wait()


@jax.jit
def kernel(x):
    mesh = plsc.VectorSubcoreMesh(core_axis_name="c", subcore_axis_name="s")
    f = functools.partial(
        pl.kernel,
        mesh=mesh,
        out_type=jax.ShapeDtypeStruct((N_ROWS, N_COLS), jnp.float32),
        scratch_types=[
            pltpu.VMEM((CHUNK, N_COLS), jnp.float32),
            pltpu.VMEM((CHUNK, N_COLS), jnp.float32),
            pltpu.VMEM((CHUNK, N_COLS), jnp.float32),
            pltpu.VMEM((CHUNK, N_COLS), jnp.float32),
            pltpu.SemaphoreType.DMA,
            pltpu.SemaphoreType.DMA,
            pltpu.SemaphoreType.DMA,
            pltpu.SemaphoreType.DMA,
        ],
        compiler_params=pltpu.CompilerParams(needs_layout_passes=False),
    )(_sc_body)
    return f(x)
